# baseline (speedup 1.0000x reference)
import jax
import jax.numpy as jnp
from jax import lax
from jax.experimental import pallas as pl
from jax.experimental.pallas import tpu as pltpu

_BB = 16
_NBUF = 2


def _fused_score(users_grp, onehot, items, table):
    B = users_grp.shape[0]
    N, D = items.shape
    nsteps = B // _BB

    def body(ug_ref, oh_ref, it_ref, tbl_ref, out_ref, acc, gbuf, osems, gsems):
        i = pl.program_id(0)
        buf = lax.rem(i, _NBUF)

        def gather_copies(p, step):
            cps = []
            for r in range(_BB):
                g = ug_ref[step * _BB + r]
                cps.append(
                    pltpu.make_async_copy(
                        tbl_ref.at[pl.ds(g * 8, 8), :],
                        gbuf.at[p, r],
                        gsems.at[p],
                    )
                )
            return cps

        def out_copy(b, j):
            return pltpu.make_async_copy(
                acc.at[b], out_ref.at[pl.ds(j * _BB, _BB), :], osems.at[b]
            )

        @pl.when(i == 0)
        def _():
            for cp in gather_copies(0, 0):
                cp.start()

        @pl.when(i + 1 < nsteps)
        def _():
            for b in range(2):

                @pl.when(lax.rem(i + 1, 2) == b)
                def _():
                    for cp in gather_copies(b, i + 1):
                        cp.start()

        for b in range(2):

            @pl.when(lax.rem(i, 2) == b)
            def _():
                for cp in gather_copies(b, i):
                    cp.wait()

        @pl.when(i >= _NBUF)
        def _():
            for b in range(_NBUF):

                @pl.when(buf == b)
                def _():
                    out_copy(b, i - _NBUF).wait()

        for b in range(2):

            @pl.when(lax.rem(i, 2) == b)
            def _():
                grp = gbuf[b]
                users_emb = jnp.sum(
                    grp * oh_ref[...][:, :, None], axis=1
                ).astype(jnp.bfloat16)
                scores_t = jax.nn.sigmoid(
                    lax.dot_general(
                        it_ref[...],
                        users_emb,
                        (((1,), (1,)), ((), ())),
                        preferred_element_type=jnp.float32,
                    )
                )
                scores = jnp.transpose(scores_t)
                for ob in range(_NBUF):

                    @pl.when(buf == ob)
                    def _():
                        acc[ob] = scores
                        out_copy(ob, i).start()

        @pl.when(i == nsteps - 1)
        def _():
            for k in range(_NBUF):
                j = nsteps - _NBUF + k
                if j >= 0:
                    out_copy(j % _NBUF, j).wait()

    return pl.pallas_call(
        body,
        grid=(nsteps,),
        in_specs=[
            pl.BlockSpec(memory_space=pltpu.SMEM),
            pl.BlockSpec((_BB, 8), lambda i: (i, 0)),
            pl.BlockSpec((N, D), lambda i: (0, 0)),
            pl.BlockSpec(memory_space=pl.ANY),
        ],
        out_specs=pl.BlockSpec(memory_space=pl.ANY),
        out_shape=jax.ShapeDtypeStruct((B, N), jnp.float32),
        scratch_shapes=[
            pltpu.VMEM((_NBUF, _BB, N), jnp.float32),
            pltpu.VMEM((2, _BB, 8, D), jnp.float32),
            pltpu.SemaphoreType.DMA((_NBUF,)),
            pltpu.SemaphoreType.DMA((2,)),
        ],
        compiler_params=pltpu.CompilerParams(
            vmem_limit_bytes=100 * 1024 * 1024,
        ),
    )(users_grp, onehot, items, table)


def kernel(users, embedding_user_weight, embedding_item_weight):
    users = users.astype(jnp.int32)
    users_grp = users // 8
    onehot = (users[:, None] % 8 == jnp.arange(8)[None, :]).astype(jnp.float32)
    return _fused_score(
        users_grp,
        onehot,
        embedding_item_weight.astype(jnp.bfloat16),
        embedding_user_weight,
    )


# speedup vs baseline: 1.5552x; 1.5552x over previous
import jax
import jax.numpy as jnp
from jax import lax
from jax.experimental import pallas as pl
from jax.experimental.pallas import tpu as pltpu

_BB = 64
_NC = 16384
_NBUF = 2


def _fused_score(users_grp, onehot, items_bf16, table):
    B = users_grp.shape[0]
    N, D = items_bf16.shape
    nbb = B // _BB
    nfull = N // _NC
    rem = N - nfull * _NC
    nc = nfull + (1 if rem else 0)

    def body(ug_ref, oh_ref, it_ref, tbl_ref, out_ref, acc, gbuf, uemb, osems, gsems):
        bb = pl.program_id(0)
        c = pl.program_id(1)
        ob = lax.rem(bb, _NBUF)

        def gather_copies(p, blk):
            cps = []
            for r in range(_BB):
                g = ug_ref[blk * _BB + r]
                cps.append(
                    pltpu.make_async_copy(
                        tbl_ref.at[pl.ds(g * 8, 8), :],
                        gbuf.at[p, r],
                        gsems.at[p],
                    )
                )
            return cps

        def out_copy(b, blk):
            return pltpu.make_async_copy(
                acc.at[b], out_ref.at[pl.ds(blk * _BB, _BB), :], osems.at[b]
            )

        @pl.when(c == 0)
        def _():
            @pl.when(bb == 0)
            def _():
                for cp in gather_copies(0, 0):
                    cp.start()

            @pl.when(bb + 1 < nbb)
            def _():
                for p in range(2):

                    @pl.when(lax.rem(bb + 1, 2) == p)
                    def _():
                        for cp in gather_copies(p, bb + 1):
                            cp.start()

            @pl.when(bb >= _NBUF)
            def _():
                for b in range(_NBUF):

                    @pl.when(ob == b)
                    def _():
                        out_copy(b, bb - _NBUF).wait()

            for p in range(2):

                @pl.when(lax.rem(bb, 2) == p)
                def _():
                    for cp in gather_copies(p, bb):
                        cp.wait()
                    grp = gbuf[p]
                    uemb[...] = jnp.sum(
                        grp * oh_ref[...][:, :, None], axis=1
                    ).astype(jnp.bfloat16)

        scores = jax.nn.sigmoid(
            lax.dot_general(
                uemb[...],
                it_ref[...],
                (((1,), (1,)), ((), ())),
                preferred_element_type=jnp.float32,
            )
        )

        for b in range(_NBUF):

            @pl.when(ob == b)
            def _():
                if rem:

                    @pl.when(c < nfull)
                    def _():
                        acc[b, :, pl.dslice(c * _NC, _NC)] = scores

                    @pl.when(c == nfull)
                    def _():
                        acc[b, :, pl.dslice(nfull * _NC, rem)] = scores[:, :rem]

                else:
                    acc[b, :, pl.dslice(c * _NC, _NC)] = scores

                @pl.when(c == nc - 1)
                def _():
                    out_copy(b, bb).start()

        @pl.when((bb == nbb - 1) & (c == nc - 1))
        def _():
            for k in range(_NBUF):
                j = nbb - _NBUF + k
                if j >= 0:
                    out_copy(j % _NBUF, j).wait()

    return pl.pallas_call(
        body,
        grid=(nbb, nc),
        in_specs=[
            pl.BlockSpec(memory_space=pltpu.SMEM),
            pl.BlockSpec((_BB, 8), lambda bb, c: (bb, 0)),
            pl.BlockSpec((_NC, D), lambda bb, c: (c, 0)),
            pl.BlockSpec(memory_space=pl.ANY),
        ],
        out_specs=pl.BlockSpec(memory_space=pl.ANY),
        out_shape=jax.ShapeDtypeStruct((B, N), jnp.float32),
        scratch_shapes=[
            pltpu.VMEM((_NBUF, _BB, N), jnp.float32),
            pltpu.VMEM((2, _BB, 8, D), jnp.float32),
            pltpu.VMEM((_BB, D), jnp.bfloat16),
            pltpu.SemaphoreType.DMA((_NBUF,)),
            pltpu.SemaphoreType.DMA((2,)),
        ],
        compiler_params=pltpu.CompilerParams(
            vmem_limit_bytes=100 * 1024 * 1024,
        ),
    )(users_grp, onehot, items_bf16, table)


def kernel(users, embedding_user_weight, embedding_item_weight):
    users = users.astype(jnp.int32)
    users_grp = users // 8
    onehot = (users[:, None] % 8 == jnp.arange(8)[None, :]).astype(jnp.float32)
    return _fused_score(
        users_grp,
        onehot,
        embedding_item_weight.astype(jnp.bfloat16),
        embedding_user_weight,
    )
